# Initial kernel scaffold; baseline (speedup 1.0000x reference)
import jax
import jax.numpy as jnp
from jax.experimental import pallas as pl


def kernel(x_user, x_item, edge_index, edge_attr, l0_u2i_W_src, l0_u2i_W_dst, l0_u2i_W_edge, l0_u2i_a_src, l0_u2i_a_dst, l0_u2i_a_edge, l0_u2i_b, l0_i2u_W_src, l0_i2u_W_dst, l0_i2u_W_edge, l0_i2u_a_src, l0_i2u_a_dst, l0_i2u_a_edge, l0_i2u_b, l1_u2i_W_src, l1_u2i_W_dst, l1_u2i_W_edge, l1_u2i_a_src, l1_u2i_a_dst, l1_u2i_a_edge, l1_u2i_b, l1_i2u_W_src, l1_i2u_W_dst, l1_i2u_W_edge, l1_i2u_a_src, l1_i2u_a_dst, l1_i2u_a_edge, l1_i2u_b, Wu, bu, Wi, bi):
    raise NotImplementedError("write your pallas kernel here")


# speedup vs baseline: 10.5084x; 10.5084x over previous
import functools
import jax
import jax.numpy as jnp
from jax.experimental import pallas as pl

_H = 2
_C = 32
_HC = _H * _C
_N = 50000
_E = 500000
_EB = 2048
_NEB = 245
_EPAD = _EB * _NEB
_R = 512
_PADDST = 50688
_ROWS = _PADDST + _R
_BM = 400


def _mm_kernel(x_ref, w_ref, o_ref):
    o_ref[...] = jnp.dot(x_ref[...], w_ref[...],
                         preferred_element_type=jnp.float32)


def _mm(x, w):
    m, k = x.shape
    n = w.shape[1]
    return pl.pallas_call(
        _mm_kernel,
        grid=(m // _BM,),
        in_specs=[pl.BlockSpec((_BM, k), lambda i: (i, 0)),
                  pl.BlockSpec((k, n), lambda i: (0, 0))],
        out_specs=pl.BlockSpec((_BM, n), lambda i: (i, 0)),
        out_shape=jax.ShapeDtypeStruct((m, n), jnp.float32),
    )(x, w)


def _edge_kernel(feat_ref, dst_ref, prm_ref, msg_ref, den_ref):
    pid = pl.program_id(0)

    @pl.when(pid == 0)
    def _init():
        msg_ref[...] = jnp.zeros_like(msg_ref)
        den_ref[...] = jnp.zeros_like(den_ref)

    feat = feat_ref[...]
    dv = dst_ref[0, :, :]
    amax0 = prm_ref[0, 0]
    amax1 = prm_ref[0, 1]

    a = feat[:, 64:66] + feat[:, 66:68]
    a = jnp.where(a >= 0, a, 0.2 * a)
    amax = jnp.concatenate(
        [jnp.full((_EB, 1), amax0, jnp.float32),
         jnp.full((_EB, 1), amax1, jnp.float32)], axis=1)
    w = jnp.exp(a - amax)
    wide = jnp.concatenate(
        [jnp.broadcast_to(w[:, 0:1], (_EB, _C)),
         jnp.broadcast_to(w[:, 1:2], (_EB, _C))], axis=1)
    msgw = feat[:, 0:_HC] * wide
    wpad = jnp.concatenate([w, jnp.zeros((_EB, 6), jnp.float32)], axis=1)

    d_last = jnp.max(dv)
    d0_init = (jnp.min(dv) // 8) * 8

    def cond(d0):
        return d0 <= d_last

    def body(d0):
        rel = dv - d0
        rows = jax.lax.broadcasted_iota(jnp.int32, (_R, _EB), 0)
        mask = (jnp.broadcast_to(rel, (_R, _EB)) == rows)
        maskf = mask.astype(jnp.float32)
        msg_ref[pl.ds(d0, _R), :] += jnp.dot(
            maskf, msgw, preferred_element_type=jnp.float32)
        den_ref[pl.ds(d0, _R), :] += jnp.dot(
            maskf, wpad, preferred_element_type=jnp.float32)
        nxt = jnp.min(jnp.where(rel >= _R, dv, jnp.int32(2 ** 30)))
        return jnp.maximum((nxt // 8) * 8, d0 + _R)

    jax.lax.while_loop(cond, body, d0_init)


def _edge_pass(feat, dst3, prm):
    return pl.pallas_call(
        _edge_kernel,
        grid=(_NEB,),
        in_specs=[pl.BlockSpec((_EB, 128), lambda i: (i, 0)),
                  pl.BlockSpec((1, 1, _EB), lambda i: (i, 0, 0)),
                  pl.BlockSpec((8, 128), lambda i: (0, 0))],
        out_specs=[pl.BlockSpec((_ROWS, _HC), lambda i: (0, 0)),
                   pl.BlockSpec((_ROWS, 8), lambda i: (0, 0))],
        out_shape=[jax.ShapeDtypeStruct((_ROWS, _HC), jnp.float32),
                   jax.ShapeDtypeStruct((_ROWS, 8), jnp.float32)],
    )(feat, dst3, prm)


def _finish_kernel(msg_ref, den_ref, b_ref, o_ref):
    den = den_ref[...]
    s = jnp.concatenate(
        [jnp.broadcast_to(den[:, 0:1], (_BM, _C)),
         jnp.broadcast_to(den[:, 1:2], (_BM, _C))], axis=1)
    r = msg_ref[...] / (s + 1e-16) + b_ref[...]
    o_ref[...] = jnp.where(r > 0, r, jnp.exp(r) - 1.0)


def _finish(msg, den, b):
    return pl.pallas_call(
        _finish_kernel,
        grid=(_N // _BM,),
        in_specs=[pl.BlockSpec((_BM, _HC), lambda i: (i, 0)),
                  pl.BlockSpec((_BM, 8), lambda i: (i, 0)),
                  pl.BlockSpec((1, _HC), lambda i: (0, 0))],
        out_specs=pl.BlockSpec((_BM, _HC), lambda i: (i, 0)),
        out_shape=jax.ShapeDtypeStruct((_N, _HC), jnp.float32),
    )(msg, den, b.reshape(1, _HC))


def _proj_kernel(x_ref, w_ref, b_ref, o_ref):
    y = jnp.dot(x_ref[...], w_ref[...],
                preferred_element_type=jnp.float32) + b_ref[...]
    nrm = jnp.sqrt(jnp.sum(y * y, axis=1, keepdims=True))
    o_ref[...] = y / jnp.maximum(nrm, 1e-12)


def _proj(x, w, b):
    n = w.shape[1]
    return pl.pallas_call(
        _proj_kernel,
        grid=(_N // _BM,),
        in_specs=[pl.BlockSpec((_BM, _HC), lambda i: (i, 0)),
                  pl.BlockSpec((_HC, n), lambda i: (0, 0)),
                  pl.BlockSpec((1, n), lambda i: (0, 0))],
        out_specs=pl.BlockSpec((_BM, n), lambda i: (i, 0)),
        out_shape=jax.ShapeDtypeStruct((_N, n), jnp.float32),
    )(x, w, b)


def _amat(a):
    z = jnp.zeros((_HC, _H), jnp.float32)
    z = z.at[0:_C, 0].set(a[0])
    z = z.at[_C:_HC, 1].set(a[1])
    return z


def _conv(table_src, table_dst, src_s, dst3, hedot_s, p):
    feat = table_src[src_s]
    auxsum = table_dst[dst3["dst_flat"], 66:68] + hedot_s
    feat = feat.at[:, 66:68].set(auxsum)
    pre = feat[:, 64:66] + auxsum
    mx = jnp.max(pre, axis=0)
    amax = jnp.where(mx >= 0, mx, 0.2 * mx)
    prm = jnp.zeros((8, 128), jnp.float32).at[0, 0:2].set(amax)
    feat = jnp.pad(feat, ((0, _EPAD - _E), (0, 0)))
    msg, den = _edge_pass(feat, dst3["dst3"], prm)
    return _finish(msg, den, p["b"])


def kernel(x_user, x_item, edge_index, edge_attr,
           l0_u2i_W_src, l0_u2i_W_dst, l0_u2i_W_edge,
           l0_u2i_a_src, l0_u2i_a_dst, l0_u2i_a_edge, l0_u2i_b,
           l0_i2u_W_src, l0_i2u_W_dst, l0_i2u_W_edge,
           l0_i2u_a_src, l0_i2u_a_dst, l0_i2u_a_edge, l0_i2u_b,
           l1_u2i_W_src, l1_u2i_W_dst, l1_u2i_W_edge,
           l1_u2i_a_src, l1_u2i_a_dst, l1_u2i_a_edge, l1_u2i_b,
           l1_i2u_W_src, l1_i2u_W_dst, l1_i2u_W_edge,
           l1_i2u_a_src, l1_i2u_a_dst, l1_i2u_a_edge, l1_i2u_b,
           Wu, bu, Wi, bi):
    prm = {}
    for nm, val in (("l0_u2i", (l0_u2i_W_src, l0_u2i_W_dst, l0_u2i_W_edge,
                                l0_u2i_a_src, l0_u2i_a_dst, l0_u2i_a_edge,
                                l0_u2i_b)),
                    ("l0_i2u", (l0_i2u_W_src, l0_i2u_W_dst, l0_i2u_W_edge,
                                l0_i2u_a_src, l0_i2u_a_dst, l0_i2u_a_edge,
                                l0_i2u_b)),
                    ("l1_u2i", (l1_u2i_W_src, l1_u2i_W_dst, l1_u2i_W_edge,
                                l1_u2i_a_src, l1_u2i_a_dst, l1_u2i_a_edge,
                                l1_u2i_b)),
                    ("l1_i2u", (l1_i2u_W_src, l1_i2u_W_dst, l1_i2u_W_edge,
                                l1_i2u_a_src, l1_i2u_a_dst, l1_i2u_a_edge,
                                l1_i2u_b))):
        W_src, W_dst, W_edge, a_src, a_dst, a_edge, b = val
        prm[nm] = {
            "W_src": W_src, "W_dst": W_dst,
            "WsA": W_src @ _amat(a_src),
            "WdA": W_dst @ _amat(a_dst),
            "WeA": W_edge @ _amat(a_edge),
            "b": b,
        }

    perm1 = jnp.argsort(edge_index[1])
    perm0 = jnp.argsort(edge_index[0])
    u2i = {
        "src": edge_index[0][perm1],
        "dst_flat": edge_index[1][perm1],
        "ea": edge_attr[perm1],
    }
    i2u = {
        "src": edge_index[1][perm0],
        "dst_flat": edge_index[0][perm0],
        "ea": edge_attr[perm0],
    }
    for d in (u2i, i2u):
        dp = jnp.concatenate(
            [d["dst_flat"],
             jnp.full((_EPAD - _E,), _PADDST, jnp.int32)])
        d["dst3"] = dp.reshape(_NEB, 1, _EB)

    xu, xi = x_user, x_item
    for l in range(2):
        pu2i = prm["l%d_u2i" % l]
        pi2u = prm["l%d_i2u" % l]
        inn = xu.shape[1]
        Wx_u = jnp.concatenate(
            [pu2i["W_src"], pu2i["WsA"], pi2u["WdA"],
             jnp.zeros((inn, 60), jnp.float32)], axis=1)
        Wx_i = jnp.concatenate(
            [pi2u["W_src"], pi2u["WsA"], pu2i["WdA"],
             jnp.zeros((inn, 60), jnp.float32)], axis=1)
        table_u = _mm(xu, Wx_u)
        table_i = _mm(xi, Wx_i)
        new_i = _conv(table_u, table_i, u2i["src"], u2i,
                      u2i["ea"] @ pu2i["WeA"], pu2i)
        new_u = _conv(table_i, table_u, i2u["src"], i2u,
                      i2u["ea"] @ pi2u["WeA"], pi2u)
        xu, xi = new_u, new_i

    Wup = jnp.pad(Wu, ((0, 0), (0, 128 - 32)))
    Wip = jnp.pad(Wi, ((0, 0), (0, 128 - 32)))
    bup = jnp.pad(bu, (0, 128 - 32)).reshape(1, 128)
    bip = jnp.pad(bi, (0, 128 - 32)).reshape(1, 128)
    ue = _proj(xu, Wup, bup)[:, :32]
    ie = _proj(xi, Wip, bip)[:, :32]
    return (ue, ie)


# speedup vs baseline: 10.6026x; 1.0090x over previous
import functools
import jax
import jax.numpy as jnp
from jax.experimental import pallas as pl

_H = 2
_C = 32
_HC = _H * _C
_N = 50000
_E = 500000
_EB = 4096
_NEB = 123
_EPAD = _EB * _NEB
_R = 512
_PADDST = 50688
_ROWS = _PADDST + _R
_BM = 400


def _mm_kernel(x_ref, w_ref, o_ref):
    o_ref[...] = jnp.dot(x_ref[...], w_ref[...],
                         preferred_element_type=jnp.float32)


def _mm(x, w):
    m, k = x.shape
    n = w.shape[1]
    return pl.pallas_call(
        _mm_kernel,
        grid=(m // _BM,),
        in_specs=[pl.BlockSpec((_BM, k), lambda i: (i, 0)),
                  pl.BlockSpec((k, n), lambda i: (0, 0))],
        out_specs=pl.BlockSpec((_BM, n), lambda i: (i, 0)),
        out_shape=jax.ShapeDtypeStruct((m, n), jnp.float32),
    )(x, w)


def _edge_kernel(feat_ref, dst_ref, prm_ref, msg_ref, den_ref):
    pid = pl.program_id(0)

    @pl.when(pid == 0)
    def _init():
        msg_ref[...] = jnp.zeros_like(msg_ref)
        den_ref[...] = jnp.zeros_like(den_ref)

    feat = feat_ref[...]
    dv = dst_ref[0, :, :]
    amax0 = prm_ref[0, 0]
    amax1 = prm_ref[0, 1]

    a = feat[:, 64:66] + feat[:, 66:68]
    a = jnp.where(a >= 0, a, 0.2 * a)
    amax = jnp.concatenate(
        [jnp.full((_EB, 1), amax0, jnp.float32),
         jnp.full((_EB, 1), amax1, jnp.float32)], axis=1)
    w = jnp.exp(a - amax)
    wide = jnp.concatenate(
        [jnp.broadcast_to(w[:, 0:1], (_EB, _C)),
         jnp.broadcast_to(w[:, 1:2], (_EB, _C))], axis=1)
    msgw = feat[:, 0:_HC] * wide
    wpad = jnp.concatenate([w, jnp.zeros((_EB, 6), jnp.float32)], axis=1)

    d_last = jnp.max(dv)
    d0_init = (jnp.min(dv) // 8) * 8

    def cond(d0):
        return d0 <= d_last

    def body(d0):
        rel = dv - d0
        rows = jax.lax.broadcasted_iota(jnp.int32, (_R, _EB), 0)
        mask = (jnp.broadcast_to(rel, (_R, _EB)) == rows)
        maskf = mask.astype(jnp.float32)
        msg_ref[pl.ds(d0, _R), :] += jnp.dot(
            maskf, msgw, preferred_element_type=jnp.float32)
        den_ref[pl.ds(d0, _R), :] += jnp.dot(
            maskf, wpad, preferred_element_type=jnp.float32)
        nxt = jnp.min(jnp.where(rel >= _R, dv, jnp.int32(2 ** 30)))
        return jnp.maximum((nxt // 8) * 8, d0 + _R)

    jax.lax.while_loop(cond, body, d0_init)


def _edge_pass(feat, dst3, prm):
    return pl.pallas_call(
        _edge_kernel,
        grid=(_NEB,),
        in_specs=[pl.BlockSpec((_EB, 128), lambda i: (i, 0)),
                  pl.BlockSpec((1, 1, _EB), lambda i: (i, 0, 0)),
                  pl.BlockSpec((8, 128), lambda i: (0, 0))],
        out_specs=[pl.BlockSpec((_ROWS, _HC), lambda i: (0, 0)),
                   pl.BlockSpec((_ROWS, 8), lambda i: (0, 0))],
        out_shape=[jax.ShapeDtypeStruct((_ROWS, _HC), jnp.float32),
                   jax.ShapeDtypeStruct((_ROWS, 8), jnp.float32)],
    )(feat, dst3, prm)


def _finish_kernel(msg_ref, den_ref, b_ref, o_ref):
    den = den_ref[...]
    s = jnp.concatenate(
        [jnp.broadcast_to(den[:, 0:1], (_BM, _C)),
         jnp.broadcast_to(den[:, 1:2], (_BM, _C))], axis=1)
    r = msg_ref[...] / (s + 1e-16) + b_ref[...]
    o_ref[...] = jnp.where(r > 0, r, jnp.exp(r) - 1.0)


def _finish(msg, den, b):
    return pl.pallas_call(
        _finish_kernel,
        grid=(_N // _BM,),
        in_specs=[pl.BlockSpec((_BM, _HC), lambda i: (i, 0)),
                  pl.BlockSpec((_BM, 8), lambda i: (i, 0)),
                  pl.BlockSpec((1, _HC), lambda i: (0, 0))],
        out_specs=pl.BlockSpec((_BM, _HC), lambda i: (i, 0)),
        out_shape=jax.ShapeDtypeStruct((_N, _HC), jnp.float32),
    )(msg, den, b.reshape(1, _HC))


def _proj_kernel(x_ref, w_ref, b_ref, o_ref):
    y = jnp.dot(x_ref[...], w_ref[...],
                preferred_element_type=jnp.float32) + b_ref[...]
    nrm = jnp.sqrt(jnp.sum(y * y, axis=1, keepdims=True))
    o_ref[...] = y / jnp.maximum(nrm, 1e-12)


def _proj(x, w, b):
    n = w.shape[1]
    return pl.pallas_call(
        _proj_kernel,
        grid=(_N // _BM,),
        in_specs=[pl.BlockSpec((_BM, _HC), lambda i: (i, 0)),
                  pl.BlockSpec((_HC, n), lambda i: (0, 0)),
                  pl.BlockSpec((1, n), lambda i: (0, 0))],
        out_specs=pl.BlockSpec((_BM, n), lambda i: (i, 0)),
        out_shape=jax.ShapeDtypeStruct((_N, n), jnp.float32),
    )(x, w, b)


def _amat(a):
    z = jnp.zeros((_HC, _H), jnp.float32)
    z = z.at[0:_C, 0].set(a[0])
    z = z.at[_C:_HC, 1].set(a[1])
    return z


def _conv(table_src, table_dst, src_s, dst3, hedot_s, p):
    feat = table_src[src_s]
    auxsum = table_dst[dst3["dst_flat"], 66:68] + hedot_s
    feat = feat.at[:, 66:68].set(auxsum)
    pre = feat[:, 64:66] + auxsum
    mx = jnp.max(pre, axis=0)
    amax = jnp.where(mx >= 0, mx, 0.2 * mx)
    prm = jnp.zeros((8, 128), jnp.float32).at[0, 0:2].set(amax)
    feat = jnp.pad(feat, ((0, _EPAD - _E), (0, 0)))
    msg, den = _edge_pass(feat, dst3["dst3"], prm)
    return _finish(msg, den, p["b"])


def kernel(x_user, x_item, edge_index, edge_attr,
           l0_u2i_W_src, l0_u2i_W_dst, l0_u2i_W_edge,
           l0_u2i_a_src, l0_u2i_a_dst, l0_u2i_a_edge, l0_u2i_b,
           l0_i2u_W_src, l0_i2u_W_dst, l0_i2u_W_edge,
           l0_i2u_a_src, l0_i2u_a_dst, l0_i2u_a_edge, l0_i2u_b,
           l1_u2i_W_src, l1_u2i_W_dst, l1_u2i_W_edge,
           l1_u2i_a_src, l1_u2i_a_dst, l1_u2i_a_edge, l1_u2i_b,
           l1_i2u_W_src, l1_i2u_W_dst, l1_i2u_W_edge,
           l1_i2u_a_src, l1_i2u_a_dst, l1_i2u_a_edge, l1_i2u_b,
           Wu, bu, Wi, bi):
    prm = {}
    for nm, val in (("l0_u2i", (l0_u2i_W_src, l0_u2i_W_dst, l0_u2i_W_edge,
                                l0_u2i_a_src, l0_u2i_a_dst, l0_u2i_a_edge,
                                l0_u2i_b)),
                    ("l0_i2u", (l0_i2u_W_src, l0_i2u_W_dst, l0_i2u_W_edge,
                                l0_i2u_a_src, l0_i2u_a_dst, l0_i2u_a_edge,
                                l0_i2u_b)),
                    ("l1_u2i", (l1_u2i_W_src, l1_u2i_W_dst, l1_u2i_W_edge,
                                l1_u2i_a_src, l1_u2i_a_dst, l1_u2i_a_edge,
                                l1_u2i_b)),
                    ("l1_i2u", (l1_i2u_W_src, l1_i2u_W_dst, l1_i2u_W_edge,
                                l1_i2u_a_src, l1_i2u_a_dst, l1_i2u_a_edge,
                                l1_i2u_b))):
        W_src, W_dst, W_edge, a_src, a_dst, a_edge, b = val
        prm[nm] = {
            "W_src": W_src, "W_dst": W_dst,
            "WsA": W_src @ _amat(a_src),
            "WdA": W_dst @ _amat(a_dst),
            "WeA": W_edge @ _amat(a_edge),
            "b": b,
        }

    perm1 = jnp.argsort(edge_index[1])
    perm0 = jnp.argsort(edge_index[0])
    u2i = {
        "src": edge_index[0][perm1],
        "dst_flat": edge_index[1][perm1],
        "ea": edge_attr[perm1],
    }
    i2u = {
        "src": edge_index[1][perm0],
        "dst_flat": edge_index[0][perm0],
        "ea": edge_attr[perm0],
    }
    for d in (u2i, i2u):
        dp = jnp.concatenate(
            [d["dst_flat"],
             jnp.full((_EPAD - _E,), _PADDST, jnp.int32)])
        d["dst3"] = dp.reshape(_NEB, 1, _EB)

    xu, xi = x_user, x_item
    for l in range(2):
        pu2i = prm["l%d_u2i" % l]
        pi2u = prm["l%d_i2u" % l]
        inn = xu.shape[1]
        Wx_u = jnp.concatenate(
            [pu2i["W_src"], pu2i["WsA"], pi2u["WdA"],
             jnp.zeros((inn, 60), jnp.float32)], axis=1)
        Wx_i = jnp.concatenate(
            [pi2u["W_src"], pi2u["WsA"], pu2i["WdA"],
             jnp.zeros((inn, 60), jnp.float32)], axis=1)
        table_u = _mm(xu, Wx_u)
        table_i = _mm(xi, Wx_i)
        new_i = _conv(table_u, table_i, u2i["src"], u2i,
                      u2i["ea"] @ pu2i["WeA"], pu2i)
        new_u = _conv(table_i, table_u, i2u["src"], i2u,
                      i2u["ea"] @ pi2u["WeA"], pi2u)
        xu, xi = new_u, new_i

    Wup = jnp.pad(Wu, ((0, 0), (0, 128 - 32)))
    Wip = jnp.pad(Wi, ((0, 0), (0, 128 - 32)))
    bup = jnp.pad(bu, (0, 128 - 32)).reshape(1, 128)
    bip = jnp.pad(bi, (0, 128 - 32)).reshape(1, 128)
    ue = _proj(xu, Wup, bup)[:, :32]
    ie = _proj(xi, Wip, bip)[:, :32]
    return (ue, ie)


# speedup vs baseline: 10.7159x; 1.0107x over previous
import functools
import jax
import jax.numpy as jnp
from jax.experimental import pallas as pl

_H = 2
_C = 32
_HC = _H * _C
_N = 50000
_E = 500000
_EB = 4096
_NEB = 123
_EPAD = _EB * _NEB
_R = 512
_PADDST = 50688
_ROWS = _PADDST + _R
_BM = 400


def _mm_kernel(x_ref, w_ref, o_ref):
    o_ref[...] = jnp.dot(x_ref[...], w_ref[...],
                         preferred_element_type=jnp.float32)


def _mm(x, w):
    m, k = x.shape
    n = w.shape[1]
    return pl.pallas_call(
        _mm_kernel,
        grid=(m // _BM,),
        in_specs=[pl.BlockSpec((_BM, k), lambda i: (i, 0)),
                  pl.BlockSpec((k, n), lambda i: (0, 0))],
        out_specs=pl.BlockSpec((_BM, n), lambda i: (i, 0)),
        out_shape=jax.ShapeDtypeStruct((m, n), jnp.float32),
    )(x, w)


def _edge_kernel(feat_ref, dst_ref, prm_ref, msg_ref, den_ref):
    pid = pl.program_id(0)

    @pl.when(pid == 0)
    def _init():
        msg_ref[...] = jnp.zeros_like(msg_ref)
        den_ref[...] = jnp.zeros_like(den_ref)

    feat = feat_ref[...]
    dv = dst_ref[0, :, :]
    amax0 = prm_ref[0, 0]
    amax1 = prm_ref[0, 1]

    a = feat[:, 64:66] + feat[:, 66:68]
    a = jnp.where(a >= 0, a, 0.2 * a)
    amax = jnp.concatenate(
        [jnp.full((_EB, 1), amax0, jnp.float32),
         jnp.full((_EB, 1), amax1, jnp.float32)], axis=1)
    w = jnp.exp(a - amax)
    wide = jnp.concatenate(
        [jnp.broadcast_to(w[:, 0:1], (_EB, _C)),
         jnp.broadcast_to(w[:, 1:2], (_EB, _C))], axis=1)
    msgw = feat[:, 0:_HC] * wide
    wpad = jnp.concatenate([w, jnp.zeros((_EB, 6), jnp.float32)], axis=1)

    d_last = jnp.max(dv)
    d0_init = (jnp.min(dv) // 8) * 8

    def cond(d0):
        return d0 <= d_last

    def body(d0):
        rel = dv - d0
        rows = jax.lax.broadcasted_iota(jnp.int32, (_R, _EB), 0)
        mask = (jnp.broadcast_to(rel, (_R, _EB)) == rows)
        maskf = mask.astype(jnp.float32)
        msg_ref[pl.ds(d0, _R), :] += jnp.dot(
            maskf, msgw, preferred_element_type=jnp.float32)
        den_ref[pl.ds(d0, _R), :] += jnp.dot(
            maskf, wpad, preferred_element_type=jnp.float32)
        nxt = jnp.min(jnp.where(rel >= _R, dv, jnp.int32(2 ** 30)))
        return jnp.maximum((nxt // 8) * 8, d0 + _R)

    jax.lax.while_loop(cond, body, d0_init)


def _edge_pass(feat, dst3, prm):
    return pl.pallas_call(
        _edge_kernel,
        grid=(_NEB,),
        in_specs=[pl.BlockSpec((_EB, 128), lambda i: (i, 0)),
                  pl.BlockSpec((1, 1, _EB), lambda i: (i, 0, 0)),
                  pl.BlockSpec((8, 128), lambda i: (0, 0))],
        out_specs=[pl.BlockSpec((_ROWS, _HC), lambda i: (0, 0)),
                   pl.BlockSpec((_ROWS, 8), lambda i: (0, 0))],
        out_shape=[jax.ShapeDtypeStruct((_ROWS, _HC), jnp.float32),
                   jax.ShapeDtypeStruct((_ROWS, 8), jnp.float32)],
    )(feat, dst3, prm)


def _finish_kernel(msg_ref, den_ref, b_ref, o_ref):
    den = den_ref[...]
    s = jnp.concatenate(
        [jnp.broadcast_to(den[:, 0:1], (_BM, _C)),
         jnp.broadcast_to(den[:, 1:2], (_BM, _C))], axis=1)
    r = msg_ref[...] / (s + 1e-16) + b_ref[...]
    o_ref[...] = jnp.where(r > 0, r, jnp.exp(r) - 1.0)


def _finish(msg, den, b):
    return pl.pallas_call(
        _finish_kernel,
        grid=(_N // _BM,),
        in_specs=[pl.BlockSpec((_BM, _HC), lambda i: (i, 0)),
                  pl.BlockSpec((_BM, 8), lambda i: (i, 0)),
                  pl.BlockSpec((1, _HC), lambda i: (0, 0))],
        out_specs=pl.BlockSpec((_BM, _HC), lambda i: (i, 0)),
        out_shape=jax.ShapeDtypeStruct((_N, _HC), jnp.float32),
    )(msg, den, b.reshape(1, _HC))


def _proj_kernel(x_ref, w_ref, b_ref, o_ref):
    y = jnp.dot(x_ref[...], w_ref[...],
                preferred_element_type=jnp.float32) + b_ref[...]
    nrm = jnp.sqrt(jnp.sum(y * y, axis=1, keepdims=True))
    o_ref[...] = y / jnp.maximum(nrm, 1e-12)


def _proj(x, w, b):
    n = w.shape[1]
    return pl.pallas_call(
        _proj_kernel,
        grid=(_N // _BM,),
        in_specs=[pl.BlockSpec((_BM, _HC), lambda i: (i, 0)),
                  pl.BlockSpec((_HC, n), lambda i: (0, 0)),
                  pl.BlockSpec((1, n), lambda i: (0, 0))],
        out_specs=pl.BlockSpec((_BM, n), lambda i: (i, 0)),
        out_shape=jax.ShapeDtypeStruct((_N, n), jnp.float32),
    )(x, w, b)


def _amat(a):
    z = jnp.zeros((_HC, _H), jnp.float32)
    z = z.at[0:_C, 0].set(a[0])
    z = z.at[_C:_HC, 1].set(a[1])
    return z


def _conv(table_src, table_dst, src_s, dst3, hedot_s, p):
    feat = table_src[src_s]
    auxsum = table_dst[dst3["dst_flat"], 66:68] + hedot_s
    feat = feat.at[:, 66:68].set(auxsum)
    pre = feat[:, 64:66] + auxsum
    mx = jnp.max(pre, axis=0)
    amax = jnp.where(mx >= 0, mx, 0.2 * mx)
    prm = jnp.zeros((8, 128), jnp.float32).at[0, 0:2].set(amax)
    feat = jnp.pad(feat, ((0, _EPAD - _E), (0, 0)))
    msg, den = _edge_pass(feat, dst3["dst3"], prm)
    return _finish(msg, den, p["b"])


def kernel(x_user, x_item, edge_index, edge_attr,
           l0_u2i_W_src, l0_u2i_W_dst, l0_u2i_W_edge,
           l0_u2i_a_src, l0_u2i_a_dst, l0_u2i_a_edge, l0_u2i_b,
           l0_i2u_W_src, l0_i2u_W_dst, l0_i2u_W_edge,
           l0_i2u_a_src, l0_i2u_a_dst, l0_i2u_a_edge, l0_i2u_b,
           l1_u2i_W_src, l1_u2i_W_dst, l1_u2i_W_edge,
           l1_u2i_a_src, l1_u2i_a_dst, l1_u2i_a_edge, l1_u2i_b,
           l1_i2u_W_src, l1_i2u_W_dst, l1_i2u_W_edge,
           l1_i2u_a_src, l1_i2u_a_dst, l1_i2u_a_edge, l1_i2u_b,
           Wu, bu, Wi, bi):
    prm = {}
    for nm, val in (("l0_u2i", (l0_u2i_W_src, l0_u2i_W_dst, l0_u2i_W_edge,
                                l0_u2i_a_src, l0_u2i_a_dst, l0_u2i_a_edge,
                                l0_u2i_b)),
                    ("l0_i2u", (l0_i2u_W_src, l0_i2u_W_dst, l0_i2u_W_edge,
                                l0_i2u_a_src, l0_i2u_a_dst, l0_i2u_a_edge,
                                l0_i2u_b)),
                    ("l1_u2i", (l1_u2i_W_src, l1_u2i_W_dst, l1_u2i_W_edge,
                                l1_u2i_a_src, l1_u2i_a_dst, l1_u2i_a_edge,
                                l1_u2i_b)),
                    ("l1_i2u", (l1_i2u_W_src, l1_i2u_W_dst, l1_i2u_W_edge,
                                l1_i2u_a_src, l1_i2u_a_dst, l1_i2u_a_edge,
                                l1_i2u_b))):
        W_src, W_dst, W_edge, a_src, a_dst, a_edge, b = val
        prm[nm] = {
            "W_src": W_src, "W_dst": W_dst,
            "WsA": W_src @ _amat(a_src),
            "WdA": W_dst @ _amat(a_dst),
            "WeA": W_edge @ _amat(a_edge),
            "b": b,
        }

    perm1 = jnp.argsort(edge_index[1], stable=False)
    perm0 = jnp.argsort(edge_index[0], stable=False)
    u2i = {
        "src": edge_index[0][perm1],
        "dst_flat": edge_index[1][perm1],
        "ea": edge_attr[perm1],
    }
    i2u = {
        "src": edge_index[1][perm0],
        "dst_flat": edge_index[0][perm0],
        "ea": edge_attr[perm0],
    }
    for d in (u2i, i2u):
        dp = jnp.concatenate(
            [d["dst_flat"],
             jnp.full((_EPAD - _E,), _PADDST, jnp.int32)])
        d["dst3"] = dp.reshape(_NEB, 1, _EB)

    xu, xi = x_user, x_item
    for l in range(2):
        pu2i = prm["l%d_u2i" % l]
        pi2u = prm["l%d_i2u" % l]
        inn = xu.shape[1]
        Wx_u = jnp.concatenate(
            [pu2i["W_src"], pu2i["WsA"], pi2u["WdA"],
             jnp.zeros((inn, 60), jnp.float32)], axis=1)
        Wx_i = jnp.concatenate(
            [pi2u["W_src"], pi2u["WsA"], pu2i["WdA"],
             jnp.zeros((inn, 60), jnp.float32)], axis=1)
        table_u = _mm(xu, Wx_u)
        table_i = _mm(xi, Wx_i)
        new_i = _conv(table_u, table_i, u2i["src"], u2i,
                      u2i["ea"] @ pu2i["WeA"], pu2i)
        new_u = _conv(table_i, table_u, i2u["src"], i2u,
                      i2u["ea"] @ pi2u["WeA"], pi2u)
        xu, xi = new_u, new_i

    Wup = jnp.pad(Wu, ((0, 0), (0, 128 - 32)))
    Wip = jnp.pad(Wi, ((0, 0), (0, 128 - 32)))
    bup = jnp.pad(bu, (0, 128 - 32)).reshape(1, 128)
    bip = jnp.pad(bi, (0, 128 - 32)).reshape(1, 128)
    ue = _proj(xu, Wup, bup)[:, :32]
    ie = _proj(xi, Wip, bip)[:, :32]
    return (ue, ie)


# speedup vs baseline: 10.8582x; 1.0133x over previous
import functools
import jax
import jax.numpy as jnp
from jax.experimental import pallas as pl

_H = 2
_C = 32
_HC = _H * _C
_N = 50000
_E = 500000
_EB = 2048
_NEB = 245
_EPAD = _EB * _NEB
_R = 256
_PADDST = 50688
_ROWS = _PADDST + _R
_BM = 400


def _mm_kernel(x_ref, w_ref, o_ref):
    o_ref[...] = jnp.dot(x_ref[...], w_ref[...],
                         preferred_element_type=jnp.float32)


def _mm(x, w):
    m, k = x.shape
    n = w.shape[1]
    return pl.pallas_call(
        _mm_kernel,
        grid=(m // _BM,),
        in_specs=[pl.BlockSpec((_BM, k), lambda i: (i, 0)),
                  pl.BlockSpec((k, n), lambda i: (0, 0))],
        out_specs=pl.BlockSpec((_BM, n), lambda i: (i, 0)),
        out_shape=jax.ShapeDtypeStruct((m, n), jnp.float32),
    )(x, w)


def _edge_kernel(feat_ref, dst_ref, prm_ref, msg_ref, den_ref):
    pid = pl.program_id(0)

    @pl.when(pid == 0)
    def _init():
        msg_ref[...] = jnp.zeros_like(msg_ref)
        den_ref[...] = jnp.zeros_like(den_ref)

    feat = feat_ref[...]
    dv = dst_ref[0, :, :]
    amax0 = prm_ref[0, 0]
    amax1 = prm_ref[0, 1]

    a = feat[:, 64:66] + feat[:, 66:68]
    a = jnp.where(a >= 0, a, 0.2 * a)
    amax = jnp.concatenate(
        [jnp.full((_EB, 1), amax0, jnp.float32),
         jnp.full((_EB, 1), amax1, jnp.float32)], axis=1)
    w = jnp.exp(a - amax)
    wide = jnp.concatenate(
        [jnp.broadcast_to(w[:, 0:1], (_EB, _C)),
         jnp.broadcast_to(w[:, 1:2], (_EB, _C))], axis=1)
    msgw = feat[:, 0:_HC] * wide
    wpad = jnp.concatenate([w, jnp.zeros((_EB, 6), jnp.float32)], axis=1)

    d_last = jnp.max(dv)
    d0_init = (jnp.min(dv) // 8) * 8

    def cond(d0):
        return d0 <= d_last

    def body(d0):
        rel = dv - d0
        rows = jax.lax.broadcasted_iota(jnp.int32, (_R, _EB), 0)
        mask = (jnp.broadcast_to(rel, (_R, _EB)) == rows)
        maskf = mask.astype(jnp.float32)
        msg_ref[pl.ds(d0, _R), :] += jnp.dot(
            maskf, msgw, preferred_element_type=jnp.float32)
        den_ref[pl.ds(d0, _R), :] += jnp.dot(
            maskf, wpad, preferred_element_type=jnp.float32)
        nxt = jnp.min(jnp.where(rel >= _R, dv, jnp.int32(2 ** 30)))
        return jnp.maximum((nxt // 8) * 8, d0 + _R)

    jax.lax.while_loop(cond, body, d0_init)


def _edge_pass(feat, dst3, prm):
    return pl.pallas_call(
        _edge_kernel,
        grid=(_NEB,),
        in_specs=[pl.BlockSpec((_EB, 128), lambda i: (i, 0)),
                  pl.BlockSpec((1, 1, _EB), lambda i: (i, 0, 0)),
                  pl.BlockSpec((8, 128), lambda i: (0, 0))],
        out_specs=[pl.BlockSpec((_ROWS, _HC), lambda i: (0, 0)),
                   pl.BlockSpec((_ROWS, 8), lambda i: (0, 0))],
        out_shape=[jax.ShapeDtypeStruct((_ROWS, _HC), jnp.float32),
                   jax.ShapeDtypeStruct((_ROWS, 8), jnp.float32)],
    )(feat, dst3, prm)


def _finish_kernel(msg_ref, den_ref, b_ref, o_ref):
    den = den_ref[...]
    s = jnp.concatenate(
        [jnp.broadcast_to(den[:, 0:1], (_BM, _C)),
         jnp.broadcast_to(den[:, 1:2], (_BM, _C))], axis=1)
    r = msg_ref[...] / (s + 1e-16) + b_ref[...]
    o_ref[...] = jnp.where(r > 0, r, jnp.exp(r) - 1.0)


def _finish(msg, den, b):
    return pl.pallas_call(
        _finish_kernel,
        grid=(_N // _BM,),
        in_specs=[pl.BlockSpec((_BM, _HC), lambda i: (i, 0)),
                  pl.BlockSpec((_BM, 8), lambda i: (i, 0)),
                  pl.BlockSpec((1, _HC), lambda i: (0, 0))],
        out_specs=pl.BlockSpec((_BM, _HC), lambda i: (i, 0)),
        out_shape=jax.ShapeDtypeStruct((_N, _HC), jnp.float32),
    )(msg, den, b.reshape(1, _HC))


def _proj_kernel(x_ref, w_ref, b_ref, o_ref):
    y = jnp.dot(x_ref[...], w_ref[...],
                preferred_element_type=jnp.float32) + b_ref[...]
    nrm = jnp.sqrt(jnp.sum(y * y, axis=1, keepdims=True))
    o_ref[...] = y / jnp.maximum(nrm, 1e-12)


def _proj(x, w, b):
    n = w.shape[1]
    return pl.pallas_call(
        _proj_kernel,
        grid=(_N // _BM,),
        in_specs=[pl.BlockSpec((_BM, _HC), lambda i: (i, 0)),
                  pl.BlockSpec((_HC, n), lambda i: (0, 0)),
                  pl.BlockSpec((1, n), lambda i: (0, 0))],
        out_specs=pl.BlockSpec((_BM, n), lambda i: (i, 0)),
        out_shape=jax.ShapeDtypeStruct((_N, n), jnp.float32),
    )(x, w, b)


def _amat(a):
    z = jnp.zeros((_HC, _H), jnp.float32)
    z = z.at[0:_C, 0].set(a[0])
    z = z.at[_C:_HC, 1].set(a[1])
    return z


def _conv(table_src, table_dst, src_s, dst3, hedot_s, p):
    feat = table_src[src_s]
    auxsum = table_dst[dst3["dst_flat"], 66:68] + hedot_s
    feat = feat.at[:, 66:68].set(auxsum)
    pre = feat[:, 64:66] + auxsum
    mx = jnp.max(pre, axis=0)
    amax = jnp.where(mx >= 0, mx, 0.2 * mx)
    prm = jnp.zeros((8, 128), jnp.float32).at[0, 0:2].set(amax)
    feat = jnp.pad(feat, ((0, _EPAD - _E), (0, 0)))
    msg, den = _edge_pass(feat, dst3["dst3"], prm)
    return _finish(msg, den, p["b"])


def kernel(x_user, x_item, edge_index, edge_attr,
           l0_u2i_W_src, l0_u2i_W_dst, l0_u2i_W_edge,
           l0_u2i_a_src, l0_u2i_a_dst, l0_u2i_a_edge, l0_u2i_b,
           l0_i2u_W_src, l0_i2u_W_dst, l0_i2u_W_edge,
           l0_i2u_a_src, l0_i2u_a_dst, l0_i2u_a_edge, l0_i2u_b,
           l1_u2i_W_src, l1_u2i_W_dst, l1_u2i_W_edge,
           l1_u2i_a_src, l1_u2i_a_dst, l1_u2i_a_edge, l1_u2i_b,
           l1_i2u_W_src, l1_i2u_W_dst, l1_i2u_W_edge,
           l1_i2u_a_src, l1_i2u_a_dst, l1_i2u_a_edge, l1_i2u_b,
           Wu, bu, Wi, bi):
    prm = {}
    for nm, val in (("l0_u2i", (l0_u2i_W_src, l0_u2i_W_dst, l0_u2i_W_edge,
                                l0_u2i_a_src, l0_u2i_a_dst, l0_u2i_a_edge,
                                l0_u2i_b)),
                    ("l0_i2u", (l0_i2u_W_src, l0_i2u_W_dst, l0_i2u_W_edge,
                                l0_i2u_a_src, l0_i2u_a_dst, l0_i2u_a_edge,
                                l0_i2u_b)),
                    ("l1_u2i", (l1_u2i_W_src, l1_u2i_W_dst, l1_u2i_W_edge,
                                l1_u2i_a_src, l1_u2i_a_dst, l1_u2i_a_edge,
                                l1_u2i_b)),
                    ("l1_i2u", (l1_i2u_W_src, l1_i2u_W_dst, l1_i2u_W_edge,
                                l1_i2u_a_src, l1_i2u_a_dst, l1_i2u_a_edge,
                                l1_i2u_b))):
        W_src, W_dst, W_edge, a_src, a_dst, a_edge, b = val
        prm[nm] = {
            "W_src": W_src, "W_dst": W_dst,
            "WsA": W_src @ _amat(a_src),
            "WdA": W_dst @ _amat(a_dst),
            "WeA": W_edge @ _amat(a_edge),
            "b": b,
        }

    perm1 = jnp.argsort(edge_index[1], stable=False)
    perm0 = jnp.argsort(edge_index[0], stable=False)
    u2i = {
        "src": edge_index[0][perm1],
        "dst_flat": edge_index[1][perm1],
        "ea": edge_attr[perm1],
    }
    i2u = {
        "src": edge_index[1][perm0],
        "dst_flat": edge_index[0][perm0],
        "ea": edge_attr[perm0],
    }
    for d in (u2i, i2u):
        dp = jnp.concatenate(
            [d["dst_flat"],
             jnp.full((_EPAD - _E,), _PADDST, jnp.int32)])
        d["dst3"] = dp.reshape(_NEB, 1, _EB)

    xu, xi = x_user, x_item
    for l in range(2):
        pu2i = prm["l%d_u2i" % l]
        pi2u = prm["l%d_i2u" % l]
        inn = xu.shape[1]
        Wx_u = jnp.concatenate(
            [pu2i["W_src"], pu2i["WsA"], pi2u["WdA"],
             jnp.zeros((inn, 60), jnp.float32)], axis=1)
        Wx_i = jnp.concatenate(
            [pi2u["W_src"], pi2u["WsA"], pu2i["WdA"],
             jnp.zeros((inn, 60), jnp.float32)], axis=1)
        table_u = _mm(xu, Wx_u)
        table_i = _mm(xi, Wx_i)
        new_i = _conv(table_u, table_i, u2i["src"], u2i,
                      u2i["ea"] @ pu2i["WeA"], pu2i)
        new_u = _conv(table_i, table_u, i2u["src"], i2u,
                      i2u["ea"] @ pi2u["WeA"], pi2u)
        xu, xi = new_u, new_i

    Wup = jnp.pad(Wu, ((0, 0), (0, 128 - 32)))
    Wip = jnp.pad(Wi, ((0, 0), (0, 128 - 32)))
    bup = jnp.pad(bu, (0, 128 - 32)).reshape(1, 128)
    bip = jnp.pad(bi, (0, 128 - 32)).reshape(1, 128)
    ue = _proj(xu, Wup, bup)[:, :32]
    ie = _proj(xi, Wip, bip)[:, :32]
    return (ue, ie)
